# baseline (speedup 1.0000x reference)
import jax
import jax.numpy as jnp
from jax.experimental import pallas as pl
from jax.experimental.pallas import tpu as pltpu

_B = 4
_D = 256
_K = 1024
_THW = 8 * 32 * 32
_NB = 512
_NBLK = _THW // _NB
_N = _B * _THW
_COMMIT = 0.25


def _vq_body(ek2_ref, e_ref, et_ref, z_ref,
             zq_ref, idx_ref, loss_ref, perp_ref,
             ssd_acc, cnt_acc):
    b = pl.program_id(0)
    j = pl.program_id(1)
    first = jnp.logical_and(b == 0, j == 0)
    last = jnp.logical_and(b == pl.num_programs(0) - 1,
                           j == pl.num_programs(1) - 1)

    @pl.when(first)
    def _init():
        ssd_acc[0, 0] = 0.0
        cnt_acc[...] = jnp.zeros_like(cnt_acc)

    z_blk = z_ref[0]
    sm2 = jnp.dot(e_ref[...], z_blk,
                  preferred_element_type=jnp.float32)
    zn2 = jnp.sum(z_blk * z_blk, axis=0, keepdims=True)
    d = (zn2 + ek2_ref[...]) + sm2

    m = jnp.min(d, axis=0, keepdims=True)
    iota = jax.lax.broadcasted_iota(jnp.int32, (_K, _NB), 0)
    idx = jnp.min(jnp.where(d == m, iota, _K),
                  axis=0, keepdims=True)
    idx_ref[0] = idx

    onehot = (iota == idx).astype(jnp.float32)
    zq = jnp.dot(et_ref[...], onehot,
                 preferred_element_type=jnp.float32)
    diff = zq - z_blk
    zq_ref[0] = z_blk + diff
    ssd_acc[0, 0] += jnp.sum(diff * diff)
    cnt_acc[...] += jnp.sum(onehot, axis=1, keepdims=True)

    @pl.when(last)
    def _fini():
        loss = (1.0 + _COMMIT) * ssd_acc[0, 0] / float(_N * _D)
        loss_ref[...] = jnp.reshape(loss, (1, 1))
        p = cnt_acc[...] * (1.0 / float(_N))
        perp = jnp.exp(-jnp.sum(p * jnp.log(p + 1e-10)))
        perp_ref[...] = jnp.reshape(perp, (1, 1))


def kernel(z, embedding):
    z3 = z.reshape(_B, _D, _THW)
    ek2 = (embedding ** 2).sum(axis=1).reshape(_K, 1)
    em2 = -2.0 * embedding
    et = embedding.T

    grid = (_B, _NBLK)
    zq3, idx3, loss, perp = pl.pallas_call(
        _vq_body,
        grid=grid,
        in_specs=[
            pl.BlockSpec((_K, 1), lambda b, j: (0, 0)),
            pl.BlockSpec((_K, _D), lambda b, j: (0, 0)),
            pl.BlockSpec((_D, _K), lambda b, j: (0, 0)),
            pl.BlockSpec((1, _D, _NB), lambda b, j: (b, 0, j)),
        ],
        out_specs=[
            pl.BlockSpec((1, _D, _NB), lambda b, j: (b, 0, j)),
            pl.BlockSpec((1, 1, _NB), lambda b, j: (b * _NBLK + j, 0, 0)),
            pl.BlockSpec((1, 1), lambda b, j: (0, 0)),
            pl.BlockSpec((1, 1), lambda b, j: (0, 0)),
        ],
        out_shape=[
            jax.ShapeDtypeStruct((_B, _D, _THW), jnp.float32),
            jax.ShapeDtypeStruct((_B * _NBLK, 1, _NB), jnp.int32),
            jax.ShapeDtypeStruct((1, 1), jnp.float32),
            jax.ShapeDtypeStruct((1, 1), jnp.float32),
        ],
        scratch_shapes=[
            pltpu.SMEM((1, 1), jnp.float32),
            pltpu.VMEM((_K, 1), jnp.float32),
        ],
        compiler_params=pltpu.CompilerParams(
            dimension_semantics=("arbitrary", "arbitrary"),
        ),
    )(ek2, em2, et, z3)

    z_q = zq3.reshape(z.shape)
    idx = idx3.reshape(_N)
    return (z_q, loss[0, 0], idx, perp[0, 0])


# speedup vs baseline: 1.1444x; 1.1444x over previous
import jax
import jax.numpy as jnp
from jax.experimental import pallas as pl
from jax.experimental.pallas import tpu as pltpu

_B = 4
_D = 256
_K = 1024
_THW = 8 * 32 * 32
_NB = 1024
_NBLK = _THW // _NB
_N = _B * _THW
_COMMIT = 0.25


def _vq_body(ek2_ref, e_ref, et_ref, z_ref,
             zq_ref, idx_ref, loss_ref, perp_ref,
             ssd_acc, cnt_acc):
    b = pl.program_id(0)
    j = pl.program_id(1)
    first = jnp.logical_and(b == 0, j == 0)
    last = jnp.logical_and(b == pl.num_programs(0) - 1,
                           j == pl.num_programs(1) - 1)

    @pl.when(first)
    def _init():
        ssd_acc[0, 0] = 0.0
        cnt_acc[...] = jnp.zeros_like(cnt_acc)

    z_blk = z_ref[0]
    sm2 = jnp.dot(e_ref[...], z_blk,
                  preferred_element_type=jnp.float32)
    zn2 = jnp.sum(z_blk * z_blk, axis=0, keepdims=True)
    d = (zn2 + ek2_ref[...]) + sm2

    m = jnp.min(d, axis=0, keepdims=True)
    iota = jax.lax.broadcasted_iota(jnp.int32, (_K, _NB), 0)
    idx = jnp.min(jnp.where(d == m, iota, _K),
                  axis=0, keepdims=True)
    idx_ref[0] = idx

    onehot = (iota == idx).astype(jnp.float32)
    zq = jnp.dot(et_ref[...], onehot,
                 preferred_element_type=jnp.float32)
    diff = zq - z_blk
    zq_ref[0] = z_blk + diff
    ssd_acc[0, 0] += jnp.sum(diff * diff)
    cnt_acc[...] += jnp.sum(onehot, axis=1, keepdims=True)

    @pl.when(last)
    def _fini():
        loss = (1.0 + _COMMIT) * ssd_acc[0, 0] / float(_N * _D)
        loss_ref[...] = jnp.reshape(loss, (1, 1))
        p = cnt_acc[...] * (1.0 / float(_N))
        perp = jnp.exp(-jnp.sum(p * jnp.log(p + 1e-10)))
        perp_ref[...] = jnp.reshape(perp, (1, 1))


def kernel(z, embedding):
    z3 = z.reshape(_B, _D, _THW)
    ek2 = (embedding ** 2).sum(axis=1).reshape(_K, 1)
    em2 = -2.0 * embedding
    et = embedding.T

    grid = (_B, _NBLK)
    zq3, idx3, loss, perp = pl.pallas_call(
        _vq_body,
        grid=grid,
        in_specs=[
            pl.BlockSpec((_K, 1), lambda b, j: (0, 0)),
            pl.BlockSpec((_K, _D), lambda b, j: (0, 0)),
            pl.BlockSpec((_D, _K), lambda b, j: (0, 0)),
            pl.BlockSpec((1, _D, _NB), lambda b, j: (b, 0, j)),
        ],
        out_specs=[
            pl.BlockSpec((1, _D, _NB), lambda b, j: (b, 0, j)),
            pl.BlockSpec((1, 1, _NB), lambda b, j: (b * _NBLK + j, 0, 0)),
            pl.BlockSpec((1, 1), lambda b, j: (0, 0)),
            pl.BlockSpec((1, 1), lambda b, j: (0, 0)),
        ],
        out_shape=[
            jax.ShapeDtypeStruct((_B, _D, _THW), jnp.float32),
            jax.ShapeDtypeStruct((_B * _NBLK, 1, _NB), jnp.int32),
            jax.ShapeDtypeStruct((1, 1), jnp.float32),
            jax.ShapeDtypeStruct((1, 1), jnp.float32),
        ],
        scratch_shapes=[
            pltpu.SMEM((1, 1), jnp.float32),
            pltpu.VMEM((_K, 1), jnp.float32),
        ],
        compiler_params=pltpu.CompilerParams(
            dimension_semantics=("arbitrary", "arbitrary"),
        ),
    )(ek2, em2, et, z3)

    z_q = zq3.reshape(z.shape)
    idx = idx3.reshape(_N)
    return (z_q, loss[0, 0], idx, perp[0, 0])


# speedup vs baseline: 1.1886x; 1.0386x over previous
import jax
import jax.numpy as jnp
from jax.experimental import pallas as pl
from jax.experimental.pallas import tpu as pltpu

_B = 4
_D = 256
_K = 1024
_THW = 8 * 32 * 32
_NB = 2048
_NBLK = _THW // _NB
_N = _B * _THW
_COMMIT = 0.25


def _vq_body(ek2_ref, e_ref, et_ref, z_ref,
             zq_ref, idx_ref, loss_ref, perp_ref,
             ssd_acc, cnt_acc):
    b = pl.program_id(0)
    j = pl.program_id(1)
    first = jnp.logical_and(b == 0, j == 0)
    last = jnp.logical_and(b == pl.num_programs(0) - 1,
                           j == pl.num_programs(1) - 1)

    @pl.when(first)
    def _init():
        ssd_acc[0, 0] = 0.0
        cnt_acc[...] = jnp.zeros_like(cnt_acc)

    z_blk = z_ref[0]
    sm2 = jnp.dot(e_ref[...], z_blk,
                  preferred_element_type=jnp.float32)
    zn2 = jnp.sum(z_blk * z_blk, axis=0, keepdims=True)
    d = (zn2 + ek2_ref[...]) + sm2

    m = jnp.min(d, axis=0, keepdims=True)
    iota = jax.lax.broadcasted_iota(jnp.int32, (_K, _NB), 0)
    idx = jnp.min(jnp.where(d == m, iota, _K),
                  axis=0, keepdims=True)
    idx_ref[0] = idx

    onehot = (iota == idx).astype(jnp.float32)
    zq = jnp.dot(et_ref[...], onehot,
                 preferred_element_type=jnp.float32)
    diff = zq - z_blk
    zq_ref[0] = z_blk + diff
    ssd_acc[0, 0] += jnp.sum(diff * diff)
    cnt_acc[...] += jnp.sum(onehot, axis=1, keepdims=True)

    @pl.when(last)
    def _fini():
        loss = (1.0 + _COMMIT) * ssd_acc[0, 0] / float(_N * _D)
        loss_ref[...] = jnp.reshape(loss, (1, 1))
        p = cnt_acc[...] * (1.0 / float(_N))
        perp = jnp.exp(-jnp.sum(p * jnp.log(p + 1e-10)))
        perp_ref[...] = jnp.reshape(perp, (1, 1))


def kernel(z, embedding):
    z3 = z.reshape(_B, _D, _THW)
    ek2 = (embedding ** 2).sum(axis=1).reshape(_K, 1)
    em2 = -2.0 * embedding
    et = embedding.T

    grid = (_B, _NBLK)
    zq3, idx3, loss, perp = pl.pallas_call(
        _vq_body,
        grid=grid,
        in_specs=[
            pl.BlockSpec((_K, 1), lambda b, j: (0, 0)),
            pl.BlockSpec((_K, _D), lambda b, j: (0, 0)),
            pl.BlockSpec((_D, _K), lambda b, j: (0, 0)),
            pl.BlockSpec((1, _D, _NB), lambda b, j: (b, 0, j)),
        ],
        out_specs=[
            pl.BlockSpec((1, _D, _NB), lambda b, j: (b, 0, j)),
            pl.BlockSpec((1, 1, _NB), lambda b, j: (b * _NBLK + j, 0, 0)),
            pl.BlockSpec((1, 1), lambda b, j: (0, 0)),
            pl.BlockSpec((1, 1), lambda b, j: (0, 0)),
        ],
        out_shape=[
            jax.ShapeDtypeStruct((_B, _D, _THW), jnp.float32),
            jax.ShapeDtypeStruct((_B * _NBLK, 1, _NB), jnp.int32),
            jax.ShapeDtypeStruct((1, 1), jnp.float32),
            jax.ShapeDtypeStruct((1, 1), jnp.float32),
        ],
        scratch_shapes=[
            pltpu.SMEM((1, 1), jnp.float32),
            pltpu.VMEM((_K, 1), jnp.float32),
        ],
        compiler_params=pltpu.CompilerParams(
            dimension_semantics=("arbitrary", "arbitrary"),
        ),
    )(ek2, em2, et, z3)

    z_q = zq3.reshape(z.shape)
    idx = idx3.reshape(_N)
    return (z_q, loss[0, 0], idx, perp[0, 0])
